# baseline (speedup 1.0000x reference)
import functools

import jax
import jax.numpy as jnp
from jax import lax
from jax.experimental import pallas as pl
from jax.experimental.pallas import tpu as pltpu
from jax.experimental.pallas import tpu_sc as plsc

B, S, D, V = 4096, 50, 64, 100000

_NC, _NS, _L = 2, 16, 16
_NW = _NC * _NS
_BH = B // 2
_BPW = _BH // _NW
_RPG = 2
_GPW = _BPW // _RPG
_NBUF = 4
_DK = D // _L


def _pool_body(x_hbm, table_hbm, out_hbm, idx_v, rows_v, pooled_v, *sems):
    wid = lax.axis_index("s") * _NC + lax.axis_index("c")
    base = wid * _BPW

    pltpu.sync_copy(x_hbm.at[wid], idx_v)

    for b in range(_NBUF):
        pltpu.async_copy(table_hbm.at[idx_v.at[b]], rows_v.at[b], sems[b])

    inv = jnp.full((_L,), 1.0 / S, dtype=jnp.float32)

    def accum(b, row):
        for k in range(_DK):
            sl = pl.ds(k * _L, _L)
            a = rows_v[b, row * S, sl]
            c = rows_v[b, row * S + 1, sl]
            for j in range(2, S, 2):
                a = a + rows_v[b, row * S + j, sl]
                c = c + rows_v[b, row * S + j + 1, sl]
            yield (a + c) * inv

    def outer(g, carry):
        for b in range(_NBUF):
            gg = g * _NBUF + b
            pltpu.make_async_copy(
                table_hbm.at[idx_v.at[0]], rows_v.at[b], sems[b]
            ).wait()
            for row in range(_RPG):
                for k, val in enumerate(accum(b, row)):
                    pooled_v[gg * _RPG + row, pl.ds(k * _L, _L)] = val

            @pl.when(gg + _NBUF < _GPW)
            def _():
                pltpu.async_copy(
                    table_hbm.at[idx_v.at[gg + _NBUF]], rows_v.at[b], sems[b]
                )
        return carry

    lax.fori_loop(0, _GPW // _NBUF, outer, 0)

    pltpu.sync_copy(pooled_v, out_hbm.at[pl.ds(base, _BPW)])


@jax.jit
def _pool(x, embed_table):
    mesh = plsc.VectorSubcoreMesh(core_axis_name="c", subcore_axis_name="s")
    kern = functools.partial(
        pl.kernel,
        mesh=mesh,
        out_type=jax.ShapeDtypeStruct((_BH, D), jnp.float32),
        scratch_types=[
            pltpu.VMEM((_GPW, _RPG * S), jnp.int32),
            pltpu.VMEM((_NBUF, _RPG * S, D), jnp.float32),
            pltpu.VMEM((_BPW, D), jnp.float32),
        ] + [pltpu.SemaphoreType.DMA] * _NBUF,
        compiler_params=pltpu.CompilerParams(use_tc_tiling_on_sc=False),
    )(_pool_body)
    return kern(x, embed_table)


_BNV = 1536


def _matmul_body(p_ref, w_ref, b_ref, o_ref):
    o_ref[...] = (
        lax.dot_general(
            w_ref[...], p_ref[...],
            dimension_numbers=(((0,), (1,)), ((), ())),
            preferred_element_type=jnp.float32,
        )
        + b_ref[...]
    )


def _matmul_body1(prev_ref, p_ref, w_ref, b_ref, o_ref):
    _matmul_body(p_ref, w_ref, b_ref, o_ref)


@jax.jit
def _matmul_t0(pooled, fc_w, fc_b2d):
    grid = (pl.cdiv(V, _BNV),)
    return pl.pallas_call(
        _matmul_body,
        grid=grid,
        in_specs=[
            pl.BlockSpec((_BH, D), lambda j: (0, 0)),
            pl.BlockSpec((D, _BNV), lambda j: (0, j)),
            pl.BlockSpec((_BNV, 1), lambda j: (j, 0)),
        ],
        out_specs=pl.BlockSpec((_BNV, _BH), lambda j: (j, 0)),
        out_shape=jax.ShapeDtypeStruct((V, B), jnp.float32),
    )(pooled, fc_w, fc_b2d)


@jax.jit
def _matmul_t1(prev, pooled, fc_w, fc_b2d):
    grid = (pl.cdiv(V, _BNV),)
    return pl.pallas_call(
        _matmul_body1,
        grid=grid,
        in_specs=[
            pl.BlockSpec(memory_space=pl.ANY),
            pl.BlockSpec((_BH, D), lambda j: (0, 0)),
            pl.BlockSpec((D, _BNV), lambda j: (0, j)),
            pl.BlockSpec((_BNV, 1), lambda j: (j, 0)),
        ],
        out_specs=pl.BlockSpec((_BNV, _BH), lambda j: (j, 1)),
        out_shape=jax.ShapeDtypeStruct((V, B), jnp.float32),
        input_output_aliases={0: 0},
    )(prev, pooled, fc_w, fc_b2d)


def kernel(x, embed_table, fc_w, fc_b):
    xr = x.astype(jnp.int32).reshape(2, _NW, _GPW, _RPG * S)
    pooled_a = _pool(xr[0], embed_table)
    pooled_b = _pool(xr[1], embed_table)
    fc_b2d = fc_b.reshape(V, 1)
    out = _matmul_t0(pooled_a, fc_w, fc_b2d)
    out = _matmul_t1(out, pooled_b, fc_w, fc_b2d)
    return out.T


# speedup vs baseline: 1.0794x; 1.0794x over previous
import functools

import jax
import jax.numpy as jnp
from jax import lax
from jax.experimental import pallas as pl
from jax.experimental.pallas import tpu as pltpu
from jax.experimental.pallas import tpu_sc as plsc

B, S, D, V = 4096, 50, 64, 100000

_NC, _NS, _L = 2, 16, 16
_NW = _NC * _NS
_BPW = B // _NW
_RPG = 2
_GPW = _BPW // _RPG
_NBUF = 4
_DK = D // _L


def _pool_body(x_hbm, table_hbm, out_hbm, idx_v, rows_v, pooled_v, *sems):
    wid = lax.axis_index("s") * _NC + lax.axis_index("c")
    base = wid * _BPW

    pltpu.sync_copy(x_hbm.at[wid], idx_v)

    for b in range(_NBUF):
        pltpu.async_copy(table_hbm.at[idx_v.at[b]], rows_v.at[b], sems[b])

    inv = jnp.full((_L,), 1.0 / S, dtype=jnp.float32)

    def accum(b, row):
        for k in range(_DK):
            sl = pl.ds(k * _L, _L)
            a = rows_v[b, row * S, sl]
            c = rows_v[b, row * S + 1, sl]
            for j in range(2, S, 2):
                a = a + rows_v[b, row * S + j, sl]
                c = c + rows_v[b, row * S + j + 1, sl]
            yield (a + c) * inv

    def outer(g, carry):
        for b in range(_NBUF):
            gg = g * _NBUF + b
            pltpu.make_async_copy(
                table_hbm.at[idx_v.at[0]], rows_v.at[b], sems[b]
            ).wait()
            for row in range(_RPG):
                for k, val in enumerate(accum(b, row)):
                    pooled_v[gg * _RPG + row, pl.ds(k * _L, _L)] = val

            @pl.when(gg + _NBUF < _GPW)
            def _():
                pltpu.async_copy(
                    table_hbm.at[idx_v.at[gg + _NBUF]], rows_v.at[b], sems[b]
                )
        return carry

    lax.fori_loop(0, _GPW // _NBUF, outer, 0)

    pltpu.sync_copy(pooled_v, out_hbm.at[pl.ds(base, _BPW)])


@jax.jit
def _pool(x, embed_table):
    mesh = plsc.VectorSubcoreMesh(core_axis_name="c", subcore_axis_name="s")
    kern = functools.partial(
        pl.kernel,
        mesh=mesh,
        out_type=jax.ShapeDtypeStruct((B, D), jnp.float32),
        scratch_types=[
            pltpu.VMEM((_GPW, _RPG * S), jnp.int32),
            pltpu.VMEM((_NBUF, _RPG * S, D), jnp.float32),
            pltpu.VMEM((_BPW, D), jnp.float32),
        ] + [pltpu.SemaphoreType.DMA] * _NBUF,
        compiler_params=pltpu.CompilerParams(use_tc_tiling_on_sc=False),
    )(_pool_body)
    return kern(x.reshape(_NW, _GPW, _RPG * S), embed_table)


_BNV = 1536


def _matmul_body(p_ref, w_ref, b_ref, o_ref):
    o_ref[...] = (
        lax.dot_general(
            w_ref[...], p_ref[...],
            dimension_numbers=(((0,), (1,)), ((), ())),
            preferred_element_type=jnp.float32,
        )
        + b_ref[...]
    )


@jax.jit
def _matmul_t(pooled, fc_w, fc_b2d):
    grid = (pl.cdiv(V, _BNV),)
    return pl.pallas_call(
        _matmul_body,
        grid=grid,
        in_specs=[
            pl.BlockSpec((B, D), lambda j: (0, 0)),
            pl.BlockSpec((D, _BNV), lambda j: (0, j)),
            pl.BlockSpec((_BNV, 1), lambda j: (j, 0)),
        ],
        out_specs=pl.BlockSpec((_BNV, B), lambda j: (j, 0)),
        out_shape=jax.ShapeDtypeStruct((V, B), jnp.float32),
    )(pooled, fc_w, fc_b2d)


def kernel(x, embed_table, fc_w, fc_b):
    pooled = _pool(x.astype(jnp.int32), embed_table)
    logits_t = _matmul_t(pooled, fc_w, fc_b.reshape(V, 1))
    return logits_t.T
